# Initial kernel scaffold; baseline (speedup 1.0000x reference)
import jax
import jax.numpy as jnp
from jax.experimental import pallas as pl


def kernel(x, Wg, W1, b1, W2, b2):
    raise NotImplementedError("write your pallas kernel here")


# speedup vs baseline: 2.0145x; 2.0145x over previous
import functools

import jax
import jax.numpy as jnp
from jax.experimental import pallas as pl

INPUT_DIM = 768
N_EXPERTS = 8
EXPERT_DIM = 128
TOP_K = 2
TILE_B = 256


def _moe_kernel(x_ref, wg_ref, w1_ref, b1_ref, w2_ref, b2_ref,
                out_ref, gw_ref):
    xt = x_ref[...]

    logits = jax.lax.dot_general(
        xt, wg_ref[...], (((1,), (1,)), ((), ())),
        preferred_element_type=jnp.float32)
    ids = jax.lax.broadcasted_iota(jnp.int32, logits.shape, 1)
    m1 = jnp.max(logits, axis=-1, keepdims=True)
    i1 = jnp.min(jnp.where(logits == m1, ids, N_EXPERTS),
                 axis=-1, keepdims=True)
    masked = jnp.where(ids == i1, -jnp.inf, logits)
    m2 = jnp.max(masked, axis=-1, keepdims=True)
    i2 = jnp.min(jnp.where(masked == m2, ids, N_EXPERTS),
                 axis=-1, keepdims=True)
    e2 = jnp.exp(m2 - m1)
    denom = 1.0 + e2
    p1 = 1.0 / denom
    p2 = e2 / denom
    gw = (jnp.where(ids == i1, p1, 0.0) +
          jnp.where(ids == i2, p2, 0.0))
    gw_ref[...] = gw

    acc = jnp.zeros((xt.shape[0], EXPERT_DIM), jnp.float32)
    for e in range(N_EXPERTS):
        h = jax.lax.dot_general(
            xt, w1_ref[e], (((1,), (1,)), ((), ())),
            preferred_element_type=jnp.float32) + b1_ref[e]
        h = 0.5 * h * (1.0 + jax.lax.erf(h * 0.7071067811865476))
        o = jax.lax.dot_general(
            h, w2_ref[e], (((1,), (1,)), ((), ())),
            preferred_element_type=jnp.float32) + b2_ref[e]
        acc = acc + gw[:, e:e + 1] * o
    out_ref[...] = acc


@functools.partial(jax.jit, static_argnames=())
def kernel(x, Wg, W1, b1, W2, b2):
    B = x.shape[0]
    grid = (B // TILE_B,)
    full = lambda *shape: pl.BlockSpec(shape, lambda i: (0,) * len(shape))
    out, gw = pl.pallas_call(
        _moe_kernel,
        grid=grid,
        in_specs=[
            pl.BlockSpec((TILE_B, INPUT_DIM), lambda i: (i, 0)),
            full(N_EXPERTS, INPUT_DIM),
            full(N_EXPERTS, EXPERT_DIM, INPUT_DIM),
            full(N_EXPERTS, EXPERT_DIM),
            full(N_EXPERTS, EXPERT_DIM, EXPERT_DIM),
            full(N_EXPERTS, EXPERT_DIM),
        ],
        out_specs=[
            pl.BlockSpec((TILE_B, EXPERT_DIM), lambda i: (i, 0)),
            pl.BlockSpec((TILE_B, N_EXPERTS), lambda i: (i, 0)),
        ],
        out_shape=[
            jax.ShapeDtypeStruct((B, EXPERT_DIM), jnp.float32),
            jax.ShapeDtypeStruct((B, N_EXPERTS), jnp.float32),
        ],
    )(x, Wg, W1, b1, W2, b2)
    return out, gw
